# baseline (device time: 573802 ns/iter reference)
import jax
import jax.numpy as jnp
from jax import lax
from jax.experimental import pallas as pl
from jax.experimental.pallas import tpu as pltpu

N_DEV = 32
N_TOK = 512
D_IN = 256
D_OUT = 512
E_PER = 2


def kernel(x, router_W, route_idx, expert_W):
    del router_W

    def body(x_ref, route_ref, w_ref, out_ref, comm_ref, send_sems, recv_sems,
             credit_sem):
        my = lax.axis_index("i")
        left = (my - 1) % N_DEV
        right = (my + 1) % N_DEV

        barrier_sem = pltpu.get_barrier_semaphore()
        for nbr in (left, right):
            pl.semaphore_signal(
                barrier_sem, inc=1,
                device_id=(nbr,), device_id_type=pl.DeviceIdType.MESH,
            )
        pl.semaphore_wait(barrier_sem, 2)

        e0 = my * E_PER
        route = route_ref[:, :]
        xv = x_ref[:, :]
        m0 = (route == e0).astype(jnp.float32)
        m1 = (route == e0 + 1).astype(jnp.float32)
        acc = jnp.dot(xv * m0, w_ref[0], preferred_element_type=jnp.float32)
        acc = acc + jnp.dot(xv * m1, w_ref[1],
                            preferred_element_type=jnp.float32)
        out_ref[:, :] = acc
        comm_ref[0, :, :] = acc

        for h in range(N_DEV - 1):
            s = h % 2
            r = (h + 1) % 2
            if h >= 1:
                pl.semaphore_wait(credit_sem, 1)
            rdma = pltpu.make_async_remote_copy(
                src_ref=comm_ref.at[s],
                dst_ref=comm_ref.at[r],
                send_sem=send_sems.at[s],
                recv_sem=recv_sems.at[r],
                device_id=(right,),
                device_id_type=pl.DeviceIdType.MESH,
            )
            rdma.start()
            rdma.wait()
            out_ref[:, :] += comm_ref[r, :, :]
            if h < N_DEV - 2:
                pl.semaphore_signal(
                    credit_sem, inc=1,
                    device_id=(left,), device_id_type=pl.DeviceIdType.MESH,
                )

    return pl.pallas_call(
        body,
        out_shape=jax.ShapeDtypeStruct((N_TOK, D_OUT), jnp.float32),
        in_specs=[
            pl.BlockSpec(memory_space=pltpu.VMEM),
            pl.BlockSpec(memory_space=pltpu.VMEM),
            pl.BlockSpec(memory_space=pltpu.VMEM),
        ],
        out_specs=pl.BlockSpec(memory_space=pltpu.VMEM),
        scratch_shapes=[
            pltpu.VMEM((2, N_TOK, D_OUT), jnp.float32),
            pltpu.SemaphoreType.DMA((2,)),
            pltpu.SemaphoreType.DMA((2,)),
            pltpu.SemaphoreType.REGULAR,
        ],
        compiler_params=pltpu.CompilerParams(collective_id=0),
    )(x, route_idx, expert_W)


# device time: 50388 ns/iter; 11.3877x vs baseline; 11.3877x over previous
import jax
import jax.numpy as jnp
from jax import lax
from jax.experimental import pallas as pl
from jax.experimental.pallas import tpu as pltpu

N_DEV = 32
N_TOK = 512
D_IN = 256
D_OUT = 512
E_PER = 2

_STAGE_OFF = (0, 256, 384, 448, 480)
_STAGE_ROWS = 496


def kernel(x, router_W, route_idx, expert_W):
    del router_W

    def body(x_ref, route_ref, w_ref, out_ref, stage_ref, send_sems,
             recv_sems):
        p = lax.axis_index("i")
        z = p // 8
        q = p % 8
        y = q // 2
        xc = (q % 2) ^ (y % 2)

        def pos(xx, yy, zz):
            return 8 * zz + 2 * yy + (xx ^ (yy % 2))

        partners = [
            pos(1 - xc, y, z),
            pos(xc, y ^ 1, z),
            pos(xc, y ^ 2, z),
            pos(xc, y, z ^ 1),
            pos(xc, y, z ^ 2),
        ]
        bits = [xc, y % 2, y // 2, z % 2, z // 2]

        barrier_sem = pltpu.get_barrier_semaphore()
        for pr in partners:
            pl.semaphore_signal(
                barrier_sem, inc=1,
                device_id=(pr,), device_id_type=pl.DeviceIdType.MESH,
            )
        pl.semaphore_wait(barrier_sem, 5)

        e0 = p * E_PER
        route = route_ref[:, :]
        xv = x_ref[:, :]
        m0 = (route == e0).astype(jnp.float32)
        m1 = (route == e0 + 1).astype(jnp.float32)
        acc = jnp.dot(xv * m0, w_ref[0], preferred_element_type=jnp.float32)
        acc = acc + jnp.dot(xv * m1, w_ref[1],
                            preferred_element_type=jnp.float32)
        out_ref[:, :] = acc

        s = jnp.int32(0)
        for k in range(5):
            half = 256 >> k
            keep = s + bits[k] * half
            send = s + (1 - bits[k]) * half
            rdma = pltpu.make_async_remote_copy(
                src_ref=out_ref.at[pl.ds(send, half), :],
                dst_ref=stage_ref.at[pl.ds(_STAGE_OFF[k], half), :],
                send_sem=send_sems.at[k],
                recv_sem=recv_sems.at[k],
                device_id=(partners[k],),
                device_id_type=pl.DeviceIdType.MESH,
            )
            rdma.start()
            rdma.wait()
            out_ref[pl.ds(keep, half), :] = (
                out_ref[pl.ds(keep, half), :]
                + stage_ref[pl.ds(_STAGE_OFF[k], half), :]
            )
            s = keep

        v = s
        cur = 16
        for k in range(4, -1, -1):
            rdma = pltpu.make_async_remote_copy(
                src_ref=out_ref.at[pl.ds(v, cur), :],
                dst_ref=out_ref.at[pl.ds(v, cur), :],
                send_sem=send_sems.at[5 + k],
                recv_sem=recv_sems.at[5 + k],
                device_id=(partners[k],),
                device_id_type=pl.DeviceIdType.MESH,
            )
            rdma.start()
            rdma.wait()
            v = v - bits[k] * cur
            cur = 2 * cur

    return pl.pallas_call(
        body,
        out_shape=jax.ShapeDtypeStruct((N_TOK, D_OUT), jnp.float32),
        in_specs=[
            pl.BlockSpec(memory_space=pltpu.VMEM),
            pl.BlockSpec(memory_space=pltpu.VMEM),
            pl.BlockSpec(memory_space=pltpu.VMEM),
        ],
        out_specs=pl.BlockSpec(memory_space=pltpu.VMEM),
        scratch_shapes=[
            pltpu.VMEM((_STAGE_ROWS, D_OUT), jnp.float32),
            pltpu.SemaphoreType.DMA((10,)),
            pltpu.SemaphoreType.DMA((10,)),
        ],
        compiler_params=pltpu.CompilerParams(collective_id=0),
    )(x, route_idx, expert_W)


# device time: 49881 ns/iter; 11.5034x vs baseline; 1.0102x over previous
import jax
import jax.numpy as jnp
from jax import lax
from jax.experimental import pallas as pl
from jax.experimental.pallas import tpu as pltpu

N_DEV = 32
N_TOK = 512
D_IN = 256
D_OUT = 512
E_PER = 2

_OFF_X, _OFF_Y1, _OFF_Y2, _OFF_Z1, _OFF_Z2 = 0, 256, 384, 448, 512
_STAGE_ROWS = 576


def kernel(x, router_W, route_idx, expert_W):
    del router_W

    def body(x_ref, route_ref, w_ref, out_ref, stage_ref, send_sems,
             recv_sems):
        p = lax.axis_index("i")
        z = p // 8
        q = p % 8
        y = q // 2
        xc = (q % 2) ^ (y % 2)

        def pos(xx, yy, zz):
            return 8 * zz + 2 * yy + (xx ^ (yy % 2))

        partner_x = pos(1 - xc, y, z)
        partner_y1 = pos(xc, y ^ 1, z)
        partner_y2 = pos(xc, y ^ 2, z)
        partner_z1 = pos(xc, y, z ^ 1)
        partner_z2 = pos(xc, y, z ^ 2)
        partners = [partner_x, partner_y1, partner_y2, partner_z1,
                    partner_z2]

        barrier_sem = pltpu.get_barrier_semaphore()
        for pr in partners:
            pl.semaphore_signal(
                barrier_sem, inc=1,
                device_id=(pr,), device_id_type=pl.DeviceIdType.MESH,
            )
        pl.semaphore_wait(barrier_sem, 5)

        e0 = p * E_PER

        def compute_half(start):
            xs = x_ref[pl.ds(start, 256), :]
            rt = route_ref[pl.ds(start, 256), :]
            m0 = (rt == e0).astype(jnp.float32)
            m1 = (rt == e0 + 1).astype(jnp.float32)
            acc = jnp.dot(xs * m0, w_ref[0],
                          preferred_element_type=jnp.float32)
            acc = acc + jnp.dot(xs * m1, w_ref[1],
                                preferred_element_type=jnp.float32)
            out_ref[pl.ds(start, 256), :] = acc

        pending = []

        def exchange(src_slice, dst_ref, sem_idx, partner):
            rdma = pltpu.make_async_remote_copy(
                src_ref=src_slice,
                dst_ref=dst_ref,
                send_sem=send_sems.at[sem_idx],
                recv_sem=recv_sems.at[sem_idx],
                device_id=(partner,),
                device_id_type=pl.DeviceIdType.MESH,
            )
            rdma.start()
            pending.append(rdma)
            return rdma

        send0 = 256 * (1 - xc)
        keep0 = 256 * xc
        compute_half(send0)
        r = exchange(out_ref.at[pl.ds(send0, 256), :],
                     stage_ref.at[pl.ds(_OFF_X, 256), :], 0, partner_x)
        compute_half(keep0)
        r.wait_recv()
        out_ref[pl.ds(keep0, 256), :] = (
            out_ref[pl.ds(keep0, 256), :]
            + stage_ref[pl.ds(_OFF_X, 256), :]
        )
        s = keep0

        for half, bit, off, sem_idx, partner in (
            (128, y % 2, _OFF_Y1, 1, partner_y1),
            (64, y // 2, _OFF_Y2, 2, partner_y2),
        ):
            keep = s + bit * half
            send = s + (1 - bit) * half
            r = exchange(out_ref.at[pl.ds(send, half), :],
                         stage_ref.at[pl.ds(off, half), :], sem_idx, partner)
            r.wait_recv()
            out_ref[pl.ds(keep, half), :] = (
                out_ref[pl.ds(keep, half), :]
                + stage_ref[pl.ds(off, half), :]
            )
            s = keep

        for off, sem_idx, partner in (
            (_OFF_Z1, 3, partner_z1),
            (_OFF_Z2, 4, partner_z2),
        ):
            r = exchange(out_ref.at[pl.ds(s, 64), :],
                         stage_ref.at[pl.ds(off, 64), :], sem_idx, partner)
            r.wait_recv()
            out_ref[pl.ds(s, 64), :] = (
                out_ref[pl.ds(s, 64), :] + stage_ref[pl.ds(off, 64), :]
            )

        v = s
        cur = 64
        for sem_idx, partner, bit in (
            (5, partner_y2, y // 2),
            (6, partner_y1, y % 2),
            (7, partner_x, xc),
        ):
            r = exchange(out_ref.at[pl.ds(v, cur), :],
                         out_ref.at[pl.ds(v, cur), :], sem_idx, partner)
            r.wait_recv()
            v = v - bit * cur
            cur = 2 * cur

        for r in pending:
            r.wait_send()

    return pl.pallas_call(
        body,
        out_shape=jax.ShapeDtypeStruct((N_TOK, D_OUT), jnp.float32),
        in_specs=[
            pl.BlockSpec(memory_space=pltpu.VMEM),
            pl.BlockSpec(memory_space=pltpu.VMEM),
            pl.BlockSpec(memory_space=pltpu.VMEM),
        ],
        out_specs=pl.BlockSpec(memory_space=pltpu.VMEM),
        scratch_shapes=[
            pltpu.VMEM((_STAGE_ROWS, D_OUT), jnp.float32),
            pltpu.SemaphoreType.DMA((8,)),
            pltpu.SemaphoreType.DMA((8,)),
        ],
        compiler_params=pltpu.CompilerParams(collective_id=0),
    )(x, route_idx, expert_W)


# device time: 34322 ns/iter; 16.7182x vs baseline; 1.4533x over previous
import jax
import jax.numpy as jnp
from jax import lax
from jax.experimental import pallas as pl
from jax.experimental.pallas import tpu as pltpu

N_DEV = 32
N_TOK = 512
D_IN = 256
D_OUT = 512
E_PER = 2

_OFF_X, _OFF_Y1, _OFF_Y2, _OFF_Z1, _OFF_Z2 = 0, 256, 384, 448, 512
_STAGE_ROWS = 576


def kernel(x, router_W, route_idx, expert_W):
    del router_W

    def body(x_ref, route_ref, w_ref, out_ref, red_ref, stage_ref,
             send_sems, recv_sems):
        p = lax.axis_index("i")
        z = p // 8
        q = p % 8
        y = q // 2
        xc = (q % 2) ^ (y % 2)

        def pos(xx, yy, zz):
            return 8 * zz + 2 * yy + (xx ^ (yy % 2))

        partner_x = pos(1 - xc, y, z)
        partner_y1 = pos(xc, y ^ 1, z)
        partner_y2 = pos(xc, y ^ 2, z)
        partner_z1 = pos(xc, y, z ^ 1)
        partner_z2 = pos(xc, y, z ^ 2)
        partners = [partner_x, partner_y1, partner_y2, partner_z1,
                    partner_z2]

        barrier_sem = pltpu.get_barrier_semaphore()
        for pr in partners:
            pl.semaphore_signal(
                barrier_sem, inc=1,
                device_id=(pr,), device_id_type=pl.DeviceIdType.MESH,
            )
        pl.semaphore_wait(barrier_sem, 5)

        e0 = p * E_PER

        def compute_half(start):
            xs = x_ref[pl.ds(start, 256), :]
            rt = route_ref[pl.ds(start, 256), :]
            m0 = (rt == e0).astype(jnp.float32)
            m1 = (rt == e0 + 1).astype(jnp.float32)
            acc = jnp.dot(xs * m0, w_ref[0],
                          preferred_element_type=jnp.float32)
            acc = acc + jnp.dot(xs * m1, w_ref[1],
                                preferred_element_type=jnp.float32)
            red_ref[pl.ds(start, 256), :] = acc.astype(jnp.bfloat16)

        pending = []

        def exchange(src_slice, dst_ref, sem_idx, partner):
            rdma = pltpu.make_async_remote_copy(
                src_ref=src_slice,
                dst_ref=dst_ref,
                send_sem=send_sems.at[sem_idx],
                recv_sem=recv_sems.at[sem_idx],
                device_id=(partner,),
                device_id_type=pl.DeviceIdType.MESH,
            )
            rdma.start()
            pending.append(rdma)
            return rdma

        send0 = 256 * (1 - xc)
        keep0 = 256 * xc
        compute_half(send0)
        r = exchange(red_ref.at[pl.ds(send0, 256), :],
                     stage_ref.at[pl.ds(_OFF_X, 256), :], 0, partner_x)
        compute_half(keep0)
        r.wait_recv()
        red_ref[pl.ds(keep0, 256), :] = (
            red_ref[pl.ds(keep0, 256), :]
            + stage_ref[pl.ds(_OFF_X, 256), :]
        )
        s = keep0

        for half, bit, off, sem_idx, partner in (
            (128, y % 2, _OFF_Y1, 1, partner_y1),
            (64, y // 2, _OFF_Y2, 2, partner_y2),
        ):
            keep = s + bit * half
            send = s + (1 - bit) * half
            r = exchange(red_ref.at[pl.ds(send, half), :],
                         stage_ref.at[pl.ds(off, half), :], sem_idx, partner)
            r.wait_recv()
            red_ref[pl.ds(keep, half), :] = (
                red_ref[pl.ds(keep, half), :]
                + stage_ref[pl.ds(off, half), :]
            )
            s = keep

        for off, sem_idx, partner in (
            (_OFF_Z1, 3, partner_z1),
            (_OFF_Z2, 4, partner_z2),
        ):
            r = exchange(red_ref.at[pl.ds(s, 64), :],
                         stage_ref.at[pl.ds(off, 64), :], sem_idx, partner)
            r.wait_recv()
            red_ref[pl.ds(s, 64), :] = (
                red_ref[pl.ds(s, 64), :] + stage_ref[pl.ds(off, 64), :]
            )

        v = s
        cur = 64
        for sem_idx, partner, bit in (
            (5, partner_y2, y // 2),
            (6, partner_y1, y % 2),
            (7, partner_x, xc),
        ):
            r = exchange(red_ref.at[pl.ds(v, cur), :],
                         red_ref.at[pl.ds(v, cur), :], sem_idx, partner)
            r.wait_recv()
            v = v - bit * cur
            cur = 2 * cur

        out_ref[:, :] = red_ref[:, :].astype(jnp.float32)

        for r in pending:
            r.wait_send()

    return pl.pallas_call(
        body,
        out_shape=jax.ShapeDtypeStruct((N_TOK, D_OUT), jnp.float32),
        in_specs=[
            pl.BlockSpec(memory_space=pltpu.VMEM),
            pl.BlockSpec(memory_space=pltpu.VMEM),
            pl.BlockSpec(memory_space=pltpu.VMEM),
        ],
        out_specs=pl.BlockSpec(memory_space=pltpu.VMEM),
        scratch_shapes=[
            pltpu.VMEM((N_TOK, D_OUT), jnp.bfloat16),
            pltpu.VMEM((_STAGE_ROWS, D_OUT), jnp.bfloat16),
            pltpu.SemaphoreType.DMA((8,)),
            pltpu.SemaphoreType.DMA((8,)),
        ],
        compiler_params=pltpu.CompilerParams(collective_id=0),
    )(x, route_idx, expert_W)


# device time: 33022 ns/iter; 17.3764x vs baseline; 1.0394x over previous
import jax
import jax.numpy as jnp
from jax import lax
from jax.experimental import pallas as pl
from jax.experimental.pallas import tpu as pltpu

N_DEV = 32
N_TOK = 512
D_IN = 256
D_OUT = 512
E_PER = 2

_OFF_X = 0
_OFF_YZ = 256
_STAGE_ROWS = 496

_SEM_XRS = 0
_SEM_YZRS = 0
_SEM_YZAG = 15
_SEM_XAG = 31


def kernel(x, router_W, route_idx, expert_W):
    del router_W

    def body(x_ref, route_ref, w_ref, out_ref, red_ref, stage_ref,
             send_sems, recv_sems):
        p = lax.axis_index("i")
        z = p // 8
        q = p % 8
        y = q // 2
        xc = (q % 2) ^ (y % 2)
        g = 4 * y + z

        def pos(xx, yy, zz):
            return 8 * zz + 2 * yy + (xx ^ (yy % 2))

        def pos_g(xx, gg):
            return pos(xx, gg // 4, gg % 4)

        partner_x = pos(1 - xc, y, z)

        barrier_sem = pltpu.get_barrier_semaphore()
        pl.semaphore_signal(
            barrier_sem, inc=1,
            device_id=(partner_x,), device_id_type=pl.DeviceIdType.MESH,
        )
        for o in range(1, 16):
            pl.semaphore_signal(
                barrier_sem, inc=1,
                device_id=(pos_g(xc, (g + o) % 16),),
                device_id_type=pl.DeviceIdType.MESH,
            )
        pl.semaphore_wait(barrier_sem, 16)

        e0 = p * E_PER

        def compute_half(start):
            xs = x_ref[pl.ds(start, 256), :]
            rt = route_ref[pl.ds(start, 256), :]
            m0 = (rt == e0).astype(jnp.float32)
            m1 = (rt == e0 + 1).astype(jnp.float32)
            acc = jnp.dot(xs * m0, w_ref[0],
                          preferred_element_type=jnp.float32)
            acc = acc + jnp.dot(xs * m1, w_ref[1],
                                preferred_element_type=jnp.float32)
            red_ref[pl.ds(start, 256), :] = acc.astype(jnp.bfloat16)

        pending = []

        def send(src_slice, dst_slice, sem_idx, partner):
            rdma = pltpu.make_async_remote_copy(
                src_ref=src_slice,
                dst_ref=dst_slice,
                send_sem=send_sems.at[sem_idx],
                recv_sem=recv_sems.at[sem_idx],
                device_id=(partner,),
                device_id_type=pl.DeviceIdType.MESH,
            )
            rdma.start()
            pending.append(rdma)
            return rdma

        col0 = 256 * xc
        send0 = 256 * (1 - xc)
        compute_half(send0)
        r_x = send(red_ref.at[pl.ds(send0, 256), :],
                   stage_ref.at[pl.ds(_OFF_X, 256), :], _SEM_XRS, partner_x)
        compute_half(col0)
        r_x.wait_recv()
        red_ref[pl.ds(col0, 256), :] = (
            red_ref[pl.ds(col0, 256), :]
            + stage_ref[pl.ds(_OFF_X, 256), :]
        )

        r_me = col0 + 16 * g
        for o in range(1, 16):
            g_dst = (g - o) % 16
            send(red_ref.at[pl.ds(col0 + 16 * g_dst, 16), :],
                 stage_ref.at[pl.ds(_OFF_YZ + 16 * (o - 1), 16), :],
                 _SEM_YZRS + o, pos_g(xc, g_dst))
        for o in range(1, 16):
            pending[o].wait_recv()
        contrib = stage_ref[pl.ds(_OFF_YZ, 240), :].reshape(15, 16, D_OUT)
        red_ref[pl.ds(r_me, 16), :] = (
            red_ref[pl.ds(r_me, 16), :] + jnp.sum(contrib, axis=0)
        )

        ag_first = len(pending)
        for o in range(1, 16):
            send(red_ref.at[pl.ds(r_me, 16), :],
                 red_ref.at[pl.ds(r_me, 16), :],
                 _SEM_YZAG + o, pos_g(xc, (g - o) % 16))
        for o in range(1, 16):
            pending[ag_first + o - 1].wait_recv()

        r_ag = send(red_ref.at[pl.ds(col0, 256), :],
                    red_ref.at[pl.ds(col0, 256), :], _SEM_XAG, partner_x)
        r_ag.wait_recv()

        out_ref[:, :] = red_ref[:, :].astype(jnp.float32)

        for r in pending:
            r.wait_send()

    return pl.pallas_call(
        body,
        out_shape=jax.ShapeDtypeStruct((N_TOK, D_OUT), jnp.float32),
        in_specs=[
            pl.BlockSpec(memory_space=pltpu.VMEM),
            pl.BlockSpec(memory_space=pltpu.VMEM),
            pl.BlockSpec(memory_space=pltpu.VMEM),
        ],
        out_specs=pl.BlockSpec(memory_space=pltpu.VMEM),
        scratch_shapes=[
            pltpu.VMEM((N_TOK, D_OUT), jnp.bfloat16),
            pltpu.VMEM((_STAGE_ROWS, D_OUT), jnp.bfloat16),
            pltpu.SemaphoreType.DMA((32,)),
            pltpu.SemaphoreType.DMA((32,)),
        ],
        compiler_params=pltpu.CompilerParams(collective_id=0),
    )(x, route_idx, expert_W)


# device time: 31330 ns/iter; 18.3148x vs baseline; 1.0540x over previous
import jax
import jax.numpy as jnp
from jax import lax
from jax.experimental import pallas as pl
from jax.experimental.pallas import tpu as pltpu

N_DEV = 32
N_TOK = 512
D_IN = 256
D_OUT = 512
E_PER = 2

_OFF_X = 0
_OFF_YZ = 256
_STAGE_ROWS = 496

_SEM_XRS = 0
_SEM_YZRS = 0
_SEM_YZAG = 15
_SEM_XFWD = 31
_N_SEMS = 47


def kernel(x, router_W, route_idx, expert_W):
    del router_W

    def body(x_ref, route_ref, w_ref, out_ref, red_ref, stage_ref,
             send_sems, recv_sems):
        p = lax.axis_index("i")
        z = p // 8
        q = p % 8
        y = q // 2
        xc = (q % 2) ^ (y % 2)
        g = 4 * y + z

        def pos(xx, yy, zz):
            return 8 * zz + 2 * yy + (xx ^ (yy % 2))

        def pos_g(xx, gg):
            return pos(xx, gg // 4, gg % 4)

        partner_x = pos(1 - xc, y, z)

        barrier_sem = pltpu.get_barrier_semaphore()
        pl.semaphore_signal(
            barrier_sem, inc=1,
            device_id=(partner_x,), device_id_type=pl.DeviceIdType.MESH,
        )
        for o in range(1, 16):
            pl.semaphore_signal(
                barrier_sem, inc=1,
                device_id=(pos_g(xc, (g + o) % 16),),
                device_id_type=pl.DeviceIdType.MESH,
            )
        pl.semaphore_wait(barrier_sem, 16)

        e0 = p * E_PER

        def compute_half(start):
            xs = x_ref[pl.ds(start, 256), :]
            rt = route_ref[pl.ds(start, 256), :]
            m0 = (rt == e0).astype(jnp.float32)
            m1 = (rt == e0 + 1).astype(jnp.float32)
            acc = jnp.dot(xs * m0, w_ref[0],
                          preferred_element_type=jnp.float32)
            acc = acc + jnp.dot(xs * m1, w_ref[1],
                                preferred_element_type=jnp.float32)
            red_ref[pl.ds(start, 256), :] = acc.astype(jnp.bfloat16)

        pending = []

        def send(src_slice, dst_slice, sem_idx, partner):
            rdma = pltpu.make_async_remote_copy(
                src_ref=src_slice,
                dst_ref=dst_slice,
                send_sem=send_sems.at[sem_idx],
                recv_sem=recv_sems.at[sem_idx],
                device_id=(partner,),
                device_id_type=pl.DeviceIdType.MESH,
            )
            rdma.start()
            pending.append(rdma)
            return rdma

        col0 = 256 * xc
        send0 = 256 * (1 - xc)
        compute_half(send0)
        r_x = send(red_ref.at[pl.ds(send0, 256), :],
                   stage_ref.at[pl.ds(_OFF_X, 256), :], _SEM_XRS, partner_x)
        compute_half(col0)
        r_x.wait_recv()
        red_ref[pl.ds(col0, 256), :] = (
            red_ref[pl.ds(col0, 256), :]
            + stage_ref[pl.ds(_OFF_X, 256), :]
        )

        r_me = col0 + 16 * g
        for o in range(1, 16):
            g_dst = (g - o) % 16
            send(red_ref.at[pl.ds(col0 + 16 * g_dst, 16), :],
                 stage_ref.at[pl.ds(_OFF_YZ + 16 * (o - 1), 16), :],
                 _SEM_YZRS + o, pos_g(xc, g_dst))
        for o in range(1, 16):
            pending[o].wait_recv()
        contrib = stage_ref[pl.ds(_OFF_YZ, 240), :].reshape(15, 16, D_OUT)
        red_ref[pl.ds(r_me, 16), :] = (
            red_ref[pl.ds(r_me, 16), :] + jnp.sum(contrib, axis=0)
        )

        fwd_descs = {}

        def fwd_block(o, gg):
            return send(red_ref.at[pl.ds(col0 + 16 * gg, 16), :],
                        red_ref.at[pl.ds(col0 + 16 * gg, 16), :],
                        _SEM_XFWD + o, partner_x)

        fwd_descs[0] = fwd_block(0, g)
        ag_first = len(pending)
        for o in range(1, 16):
            send(red_ref.at[pl.ds(r_me, 16), :],
                 red_ref.at[pl.ds(r_me, 16), :],
                 _SEM_YZAG + o, pos_g(xc, (g - o) % 16))
        for o in range(1, 16):
            pending[ag_first + o - 1].wait_recv()
            fwd_descs[o] = fwd_block(o, (g + o) % 16)

        for o in range(16):
            fwd_descs[o].wait_recv()

        out_ref[:, :] = red_ref[:, :].astype(jnp.float32)

        for r in pending:
            r.wait_send()

    return pl.pallas_call(
        body,
        out_shape=jax.ShapeDtypeStruct((N_TOK, D_OUT), jnp.float32),
        in_specs=[
            pl.BlockSpec(memory_space=pltpu.VMEM),
            pl.BlockSpec(memory_space=pltpu.VMEM),
            pl.BlockSpec(memory_space=pltpu.VMEM),
        ],
        out_specs=pl.BlockSpec(memory_space=pltpu.VMEM),
        scratch_shapes=[
            pltpu.VMEM((N_TOK, D_OUT), jnp.bfloat16),
            pltpu.VMEM((_STAGE_ROWS, D_OUT), jnp.bfloat16),
            pltpu.SemaphoreType.DMA((_N_SEMS,)),
            pltpu.SemaphoreType.DMA((_N_SEMS,)),
        ],
        compiler_params=pltpu.CompilerParams(collective_id=0),
    )(x, route_idx, expert_W)


# device time: 31020 ns/iter; 18.4978x vs baseline; 1.0100x over previous
import jax
import jax.numpy as jnp
from jax import lax
from jax.experimental import pallas as pl
from jax.experimental.pallas import tpu as pltpu

N_DEV = 32
N_TOK = 512
D_IN = 256
D_OUT = 512
E_PER = 2

_OFF_X = 0
_OFF_YZ = 256
_STAGE_ROWS = 496

_SEM_XRS = 0
_SEM_YZRS = 0
_SEM_YZAG = 15
_SEM_XFWD = 31
_N_SEMS = 47


def kernel(x, router_W, route_idx, expert_W):
    del router_W

    def body(x_ref, route_ref, w_ref, out_ref, red_ref, stage_ref,
             send_sems, recv_sems):
        p = lax.axis_index("i")
        z = p // 8
        q = p % 8
        y = q // 2
        xc = (q % 2) ^ (y % 2)
        g = 4 * y + z

        def pos(xx, yy, zz):
            return 8 * zz + 2 * yy + (xx ^ (yy % 2))

        def pos_g(xx, gg):
            return pos(xx, gg // 4, gg % 4)

        partner_x = pos(1 - xc, y, z)

        barrier_sem = pltpu.get_barrier_semaphore()
        pl.semaphore_signal(
            barrier_sem, inc=1,
            device_id=(partner_x,), device_id_type=pl.DeviceIdType.MESH,
        )
        for o in range(1, 16):
            pl.semaphore_signal(
                barrier_sem, inc=1,
                device_id=(pos_g(xc, (g + o) % 16),),
                device_id_type=pl.DeviceIdType.MESH,
            )

        e0 = p * E_PER

        def compute_half(start):
            xs = x_ref[pl.ds(start, 256), :]
            rt = route_ref[pl.ds(start, 256), :]
            m0 = (rt == e0).astype(jnp.float32)
            m1 = (rt == e0 + 1).astype(jnp.float32)
            acc = jnp.dot(xs * m0, w_ref[0],
                          preferred_element_type=jnp.float32)
            acc = acc + jnp.dot(xs * m1, w_ref[1],
                                preferred_element_type=jnp.float32)
            red_ref[pl.ds(start, 256), :] = acc.astype(jnp.bfloat16)

        pending = []

        def send(src_slice, dst_slice, sem_idx, partner):
            rdma = pltpu.make_async_remote_copy(
                src_ref=src_slice,
                dst_ref=dst_slice,
                send_sem=send_sems.at[sem_idx],
                recv_sem=recv_sems.at[sem_idx],
                device_id=(partner,),
                device_id_type=pl.DeviceIdType.MESH,
            )
            rdma.start()
            pending.append(rdma)
            return rdma

        col0 = 256 * xc
        send0 = 256 * (1 - xc)
        compute_half(send0)
        pl.semaphore_wait(barrier_sem, 16)
        r_x = send(red_ref.at[pl.ds(send0, 256), :],
                   stage_ref.at[pl.ds(_OFF_X, 256), :], _SEM_XRS, partner_x)
        compute_half(col0)
        r_x.wait_recv()
        red_ref[pl.ds(col0, 256), :] = (
            red_ref[pl.ds(col0, 256), :]
            + stage_ref[pl.ds(_OFF_X, 256), :]
        )

        r_me = col0 + 16 * g
        for o in range(1, 16):
            g_dst = (g - o) % 16
            send(red_ref.at[pl.ds(col0 + 16 * g_dst, 16), :],
                 stage_ref.at[pl.ds(_OFF_YZ + 16 * (o - 1), 16), :],
                 _SEM_YZRS + o, pos_g(xc, g_dst))
        for o in range(1, 16):
            pending[o].wait_recv()
        contrib = stage_ref[pl.ds(_OFF_YZ, 240), :].reshape(15, 16, D_OUT)
        red_ref[pl.ds(r_me, 16), :] = (
            red_ref[pl.ds(r_me, 16), :] + jnp.sum(contrib, axis=0)
        )

        fwd_descs = {}

        def fwd_block(o, gg):
            return send(red_ref.at[pl.ds(col0 + 16 * gg, 16), :],
                        red_ref.at[pl.ds(col0 + 16 * gg, 16), :],
                        _SEM_XFWD + o, partner_x)

        fwd_descs[0] = fwd_block(0, g)
        ag_first = len(pending)
        for o in range(1, 16):
            send(red_ref.at[pl.ds(r_me, 16), :],
                 red_ref.at[pl.ds(r_me, 16), :],
                 _SEM_YZAG + o, pos_g(xc, (g - o) % 16))
        for o in range(1, 16):
            pending[ag_first + o - 1].wait_recv()
            fwd_descs[o] = fwd_block(o, (g + o) % 16)

        for o in range(16):
            fwd_descs[o].wait_recv()

        out_ref[:, :] = red_ref[:, :].astype(jnp.float32)

        for r in pending:
            r.wait_send()

    return pl.pallas_call(
        body,
        out_shape=jax.ShapeDtypeStruct((N_TOK, D_OUT), jnp.float32),
        in_specs=[
            pl.BlockSpec(memory_space=pltpu.VMEM),
            pl.BlockSpec(memory_space=pltpu.VMEM),
            pl.BlockSpec(memory_space=pltpu.VMEM),
        ],
        out_specs=pl.BlockSpec(memory_space=pltpu.VMEM),
        scratch_shapes=[
            pltpu.VMEM((N_TOK, D_OUT), jnp.bfloat16),
            pltpu.VMEM((_STAGE_ROWS, D_OUT), jnp.bfloat16),
            pltpu.SemaphoreType.DMA((_N_SEMS,)),
            pltpu.SemaphoreType.DMA((_N_SEMS,)),
        ],
        compiler_params=pltpu.CompilerParams(collective_id=0),
    )(x, route_idx, expert_W)


# device time: 30971 ns/iter; 18.5271x vs baseline; 1.0016x over previous
import jax
import jax.numpy as jnp
from jax import lax
from jax.experimental import pallas as pl
from jax.experimental.pallas import tpu as pltpu

N_DEV = 32
N_TOK = 512
D_IN = 256
D_OUT = 512
E_PER = 2

_OFF_X = 0
_OFF_YZ = 256
_STAGE_ROWS = 496

_SEM_XRS_A = 0
_SEM_XRS_B = 47
_SEM_YZRS = 0
_SEM_YZAG = 15
_SEM_XFWD = 31
_N_SEMS = 48


def kernel(x, router_W, route_idx, expert_W):
    del router_W

    def body(x_ref, route_ref, w_ref, out_ref, red_ref, stage_ref,
             send_sems, recv_sems):
        p = lax.axis_index("i")
        z = p // 8
        q = p % 8
        y = q // 2
        xc = (q % 2) ^ (y % 2)
        g = 4 * y + z

        def pos(xx, yy, zz):
            return 8 * zz + 2 * yy + (xx ^ (yy % 2))

        def pos_g(xx, gg):
            return pos(xx, gg // 4, gg % 4)

        partner_x = pos(1 - xc, y, z)

        barrier_sem = pltpu.get_barrier_semaphore()
        pl.semaphore_signal(
            barrier_sem, inc=1,
            device_id=(partner_x,), device_id_type=pl.DeviceIdType.MESH,
        )
        for o in range(1, 16):
            pl.semaphore_signal(
                barrier_sem, inc=1,
                device_id=(pos_g(xc, (g + o) % 16),),
                device_id_type=pl.DeviceIdType.MESH,
            )

        e0 = p * E_PER

        def compute_rows(start, nrows):
            xs = x_ref[pl.ds(start, nrows), :]
            rt = route_ref[pl.ds(start, nrows), :]
            m0 = (rt == e0).astype(jnp.float32)
            m1 = (rt == e0 + 1).astype(jnp.float32)
            acc = jnp.dot(xs * m0, w_ref[0],
                          preferred_element_type=jnp.float32)
            acc = acc + jnp.dot(xs * m1, w_ref[1],
                                preferred_element_type=jnp.float32)
            red_ref[pl.ds(start, nrows), :] = acc.astype(jnp.bfloat16)

        pending = []

        def send(src_slice, dst_slice, sem_idx, partner):
            rdma = pltpu.make_async_remote_copy(
                src_ref=src_slice,
                dst_ref=dst_slice,
                send_sem=send_sems.at[sem_idx],
                recv_sem=recv_sems.at[sem_idx],
                device_id=(partner,),
                device_id_type=pl.DeviceIdType.MESH,
            )
            rdma.start()
            pending.append(rdma)
            return rdma

        col0 = 256 * xc
        send0 = 256 * (1 - xc)
        compute_rows(send0, 128)
        pl.semaphore_wait(barrier_sem, 16)
        r_xa = send(red_ref.at[pl.ds(send0, 128), :],
                    stage_ref.at[pl.ds(_OFF_X, 128), :],
                    _SEM_XRS_A, partner_x)
        compute_rows(send0 + 128, 128)
        r_xb = send(red_ref.at[pl.ds(send0 + 128, 128), :],
                    stage_ref.at[pl.ds(_OFF_X + 128, 128), :],
                    _SEM_XRS_B, partner_x)
        compute_rows(col0, 256)

        r_me = col0 + 16 * g
        scatter = []
        for o in range(1, 16):
            g_dst = (g - o) % 16
            scatter.append(pltpu.make_async_remote_copy(
                src_ref=red_ref.at[pl.ds(col0 + 16 * g_dst, 16), :],
                dst_ref=stage_ref.at[pl.ds(_OFF_YZ + 16 * (o - 1), 16), :],
                send_sem=send_sems.at[_SEM_YZRS + o],
                recv_sem=recv_sems.at[_SEM_YZRS + o],
                device_id=(pos_g(xc, g_dst),),
                device_id_type=pl.DeviceIdType.MESH,
            ))
        pending.extend(scatter)

        r_xa.wait_recv()
        red_ref[pl.ds(col0, 128), :] = (
            red_ref[pl.ds(col0, 128), :]
            + stage_ref[pl.ds(_OFF_X, 128), :]
        )
        for o in range(1, 16):
            g_dst = (g - o) % 16

            @pl.when(g_dst < 8)
            def _(rd=scatter[o - 1]):
                rd.start()

        r_xb.wait_recv()
        red_ref[pl.ds(col0 + 128, 128), :] = (
            red_ref[pl.ds(col0 + 128, 128), :]
            + stage_ref[pl.ds(_OFF_X + 128, 128), :]
        )
        for o in range(1, 16):
            g_dst = (g - o) % 16

            @pl.when(g_dst >= 8)
            def _(rd=scatter[o - 1]):
                rd.start()

        for rd in scatter:
            rd.wait_recv()
        contrib = stage_ref[pl.ds(_OFF_YZ, 240), :].reshape(15, 16, D_OUT)
        red_ref[pl.ds(r_me, 16), :] = (
            red_ref[pl.ds(r_me, 16), :] + jnp.sum(contrib, axis=0)
        )

        fwd_descs = {}

        def fwd_block(o, gg):
            return send(red_ref.at[pl.ds(col0 + 16 * gg, 16), :],
                        red_ref.at[pl.ds(col0 + 16 * gg, 16), :],
                        _SEM_XFWD + o, partner_x)

        fwd_descs[0] = fwd_block(0, g)
        ag_first = len(pending)
        for o in range(1, 16):
            send(red_ref.at[pl.ds(r_me, 16), :],
                 red_ref.at[pl.ds(r_me, 16), :],
                 _SEM_YZAG + o, pos_g(xc, (g - o) % 16))
        for o in range(1, 16):
            pending[ag_first + o - 1].wait_recv()
            fwd_descs[o] = fwd_block(o, (g + o) % 16)

        for o in range(16):
            fwd_descs[o].wait_recv()

        out_ref[:, :] = red_ref[:, :].astype(jnp.float32)

        for r in pending:
            r.wait_send()

    return pl.pallas_call(
        body,
        out_shape=jax.ShapeDtypeStruct((N_TOK, D_OUT), jnp.float32),
        in_specs=[
            pl.BlockSpec(memory_space=pltpu.VMEM),
            pl.BlockSpec(memory_space=pltpu.VMEM),
            pl.BlockSpec(memory_space=pltpu.VMEM),
        ],
        out_specs=pl.BlockSpec(memory_space=pltpu.VMEM),
        scratch_shapes=[
            pltpu.VMEM((N_TOK, D_OUT), jnp.bfloat16),
            pltpu.VMEM((_STAGE_ROWS, D_OUT), jnp.bfloat16),
            pltpu.SemaphoreType.DMA((_N_SEMS,)),
            pltpu.SemaphoreType.DMA((_N_SEMS,)),
        ],
        compiler_params=pltpu.CompilerParams(collective_id=0),
    )(x, route_idx, expert_W)


# device time: 27640 ns/iter; 20.7598x vs baseline; 1.1205x over previous
import jax
import jax.numpy as jnp
from jax import lax
from jax.experimental import pallas as pl
from jax.experimental.pallas import tpu as pltpu

N_DEV = 32
N_TOK = 512
D_IN = 256
D_OUT = 512
E_PER = 2

_OFF_X = 0
_OFF_YZ = 256
_STAGE_ROWS = 496

_SEM_XRS = 0
_SEM_YZRS = 0
_SEM_YZAG = 15
_SEM_XFWD = 31
_N_SEMS = 47


def kernel(x, router_W, route_idx, expert_W):
    del router_W

    def body(x_ref, route_ref, w_ref, out_ref, red_ref, stage_ref,
             send_sems, recv_sems):
        p = lax.axis_index("i")
        z = p // 8
        q = p % 8
        y = q // 2
        xc = (q % 2) ^ (y % 2)
        g = 4 * y + z

        def pos(xx, yy, zz):
            return 8 * zz + 2 * yy + (xx ^ (yy % 2))

        def pos_g(xx, gg):
            return pos(xx, gg // 4, gg % 4)

        partner_x = pos(1 - xc, y, z)

        barrier_sem = pltpu.get_barrier_semaphore()
        pl.semaphore_signal(
            barrier_sem, inc=1,
            device_id=(partner_x,), device_id_type=pl.DeviceIdType.MESH,
        )
        for o in range(1, 16):
            pl.semaphore_signal(
                barrier_sem, inc=1,
                device_id=(pos_g(xc, (g + o) % 16),),
                device_id_type=pl.DeviceIdType.MESH,
            )

        e0 = p * E_PER

        def compute_half(start):
            xs = x_ref[pl.ds(start, 256), :]
            rt = route_ref[pl.ds(start, 256), :]
            m0 = (rt == e0).astype(jnp.float32)
            m1 = (rt == e0 + 1).astype(jnp.float32)
            acc = jnp.dot(xs * m0, w_ref[0],
                          preferred_element_type=jnp.float32)
            acc = acc + jnp.dot(xs * m1, w_ref[1],
                                preferred_element_type=jnp.float32)
            red_ref[pl.ds(start, 256), :] = acc.astype(jnp.bfloat16)

        pending = []

        def send(src_slice, dst_slice, sem_idx, partner):
            rdma = pltpu.make_async_remote_copy(
                src_ref=src_slice,
                dst_ref=dst_slice,
                send_sem=send_sems.at[sem_idx],
                recv_sem=recv_sems.at[sem_idx],
                device_id=(partner,),
                device_id_type=pl.DeviceIdType.MESH,
            )
            rdma.start()
            pending.append(rdma)
            return rdma

        col0 = 256 * xc
        send0 = 256 * (1 - xc)
        compute_half(send0)
        pl.semaphore_wait(barrier_sem, 16)
        r_x = send(red_ref.at[pl.ds(send0, 256), :],
                   stage_ref.at[pl.ds(_OFF_X, 256), :], _SEM_XRS, partner_x)
        compute_half(col0)
        r_x.wait_recv()
        red_ref[pl.ds(col0, 256), :] = (
            red_ref[pl.ds(col0, 256), :]
            + stage_ref[pl.ds(_OFF_X, 256), :]
        )

        r_me = col0 + 16 * g
        for o in range(1, 16):
            g_dst = (g - o) % 16
            send(red_ref.at[pl.ds(col0 + 16 * g_dst, 16), :],
                 stage_ref.at[pl.ds(_OFF_YZ + 16 * (o - 1), 16), :],
                 _SEM_YZRS + o, pos_g(xc, g_dst))
        for o in range(1, 16):
            pending[o].wait_recv()
        contrib = stage_ref[pl.ds(_OFF_YZ, 240), :].reshape(15, 16, D_OUT)
        red_ref[pl.ds(r_me, 16), :] = (
            red_ref[pl.ds(r_me, 16), :] + jnp.sum(contrib, axis=0)
        )

        fwd_descs = {}

        def fwd_block(o, gg):
            return send(red_ref.at[pl.ds(col0 + 16 * gg, 16), :],
                        red_ref.at[pl.ds(col0 + 16 * gg, 16), :],
                        _SEM_XFWD + o, partner_x)

        fwd_descs[0] = fwd_block(0, g)
        ag_first = len(pending)
        for o in range(1, 16):
            send(red_ref.at[pl.ds(r_me, 16), :],
                 red_ref.at[pl.ds(r_me, 16), :],
                 _SEM_YZAG + o, pos_g(xc, (g - o) % 16))
        for o in range(1, 16):
            pending[ag_first + o - 1].wait_recv()
            fwd_descs[o] = fwd_block(o, (g + o) % 16)

        for o in range(16):
            fwd_descs[o].wait_recv()

        out_ref[:, :] = red_ref[:, :].astype(jnp.float32)

        for r in pending:
            r.wait_send()

    return pl.pallas_call(
        body,
        out_shape=jax.ShapeDtypeStruct((N_TOK, D_OUT), jnp.float32),
        in_specs=[
            pl.BlockSpec(memory_space=pltpu.VMEM),
            pl.BlockSpec(memory_space=pltpu.VMEM),
            pl.BlockSpec(memory_space=pltpu.VMEM),
        ],
        out_specs=pl.BlockSpec(memory_space=pltpu.VMEM),
        scratch_shapes=[
            pltpu.VMEM((N_TOK, D_OUT), jnp.bfloat16),
            pltpu.VMEM((_STAGE_ROWS, D_OUT), jnp.bfloat16),
            pltpu.SemaphoreType.DMA((_N_SEMS,)),
            pltpu.SemaphoreType.DMA((_N_SEMS,)),
        ],
        compiler_params=pltpu.CompilerParams(collective_id=0),
    )(x, route_idx, expert_W)
